# Initial kernel scaffold; baseline (speedup 1.0000x reference)
import jax
import jax.numpy as jnp
from jax.experimental import pallas as pl


def kernel(x, edge_index, node_ids, label_inds, W1, b1, W2, b2):
    raise NotImplementedError("write your pallas kernel here")


# speedup vs baseline: 17.7577x; 17.7577x over previous
import functools

import jax
import jax.numpy as jnp
from jax import lax
from jax.experimental import pallas as pl
from jax.experimental.pallas import tpu as pltpu
from jax.experimental.pallas import tpu_sc as plsc

N = 10000
E = 320000
D = 128
DL = 16
NIDS = 2000

NC = 2
NS = 16
NW = NC * NS

NPAD = 10240
SR = NPAD // NS
CH = 128
NCHUNK = 79
EPT = NCHUNK * CH
EPAD = EPT * NW

IPAD = 2048
IPT = IPAD // NW

ROWB = 1024

_mesh = plsc.VectorSubcoreMesh(
    core_axis_name="c", subcore_axis_name="s", num_cores=NC, num_subcores=NS)
_sc_params = pltpu.CompilerParams(use_tc_tiling_on_sc=False)

_f32 = jnp.float32
_i32 = jnp.int32


def _deg_body(dst3, z16, ones_h, out, dstv, onesv, degS):
    c = lax.axis_index("c")
    s = lax.axis_index("s")
    w = c * NS + s
    pltpu.sync_copy(dst3.at[w], dstv)
    pltpu.sync_copy(ones_h, onesv)
    pltpu.sync_copy(z16.at[pl.ds(s * SR, SR)], degS.at[pl.ds(s * SR, SR)])
    plsc.subcore_barrier()

    def body(j, carry):
        pltpu.sync_copy(onesv, degS.at[dstv.at[j]], add=True)
        return carry

    lax.fori_loop(0, NCHUNK, body, 0)
    plsc.subcore_barrier()
    pltpu.sync_copy(degS.at[pl.ds(s * SR, SR)], out.at[c, pl.ds(s * SR, SR)])


_deg_call = pl.kernel(
    _deg_body,
    out_type=jax.ShapeDtypeStruct((NC, NPAD, DL), _f32),
    mesh=_mesh,
    compiler_params=_sc_params,
    scratch_types=[
        pltpu.VMEM((NCHUNK, CH), _i32),
        pltpu.VMEM((CH, DL), _f32),
        pltpu.VMEM_SHARED((NPAD, DL), _f32),
    ],
)


def _acc_body(Dd, src3, dst3, g_h, z_h, out, srcv, dstv, buf, accS):
    c = lax.axis_index("c")
    s = lax.axis_index("s")
    w = c * NS + s
    pltpu.sync_copy(src3.at[w], srcv)
    pltpu.sync_copy(dst3.at[w], dstv)
    pltpu.sync_copy(z_h.at[pl.ds(s * SR, SR)], accS.at[pl.ds(s * SR, SR)])
    plsc.subcore_barrier()

    def body(j, carry):
        pltpu.sync_copy(g_h.at[srcv.at[j]], buf)
        pltpu.sync_copy(buf, accS.at[dstv.at[j]], add=True)
        return carry

    lax.fori_loop(0, NCHUNK, body, 0)
    plsc.subcore_barrier()
    pltpu.sync_copy(accS.at[pl.ds(s * SR, SR)], out.at[c, pl.ds(s * SR, SR)])


def _make_acc_call(Dd):
    return pl.kernel(
        functools.partial(_acc_body, Dd),
        out_type=jax.ShapeDtypeStruct((NC, NPAD, Dd), _f32),
        mesh=_mesh,
        compiler_params=_sc_params,
        scratch_types=[
            pltpu.VMEM((NCHUNK, CH), _i32),
            pltpu.VMEM((NCHUNK, CH), _i32),
            pltpu.VMEM((CH, Dd), _f32),
            pltpu.VMEM_SHARED((NPAD, Dd), _f32),
        ],
    )


_acc128_call = _make_acc_call(D)
_acc16_call = _make_acc_call(DL)


def _head_body(ids2, a_h, b_h, g_h, dv_h, lab_h,
               ya, yb, yg, ydv, ylab,
               idsv, bufa, bufb, bufg, bufd, bufl):
    c = lax.axis_index("c")
    s = lax.axis_index("s")
    w = c * NS + s
    pltpu.sync_copy(ids2.at[w], idsv)
    pltpu.sync_copy(a_h.at[idsv], bufa)
    pltpu.sync_copy(b_h.at[idsv], bufb)
    pltpu.sync_copy(g_h.at[idsv], bufg)
    pltpu.sync_copy(dv_h.at[idsv], bufd)
    pltpu.sync_copy(lab_h.at[idsv], bufl)
    sl = pl.ds(w * IPT, IPT)
    pltpu.sync_copy(bufa, ya.at[sl])
    pltpu.sync_copy(bufb, yb.at[sl])
    pltpu.sync_copy(bufg, yg.at[sl])
    pltpu.sync_copy(bufd, ydv.at[sl])
    pltpu.sync_copy(bufl, ylab.at[sl])


_head_call = pl.kernel(
    _head_body,
    out_type=(
        jax.ShapeDtypeStruct((IPAD, DL), _f32),
        jax.ShapeDtypeStruct((IPAD, DL), _f32),
        jax.ShapeDtypeStruct((IPAD, DL), _f32),
        jax.ShapeDtypeStruct((IPAD, DL), _f32),
        jax.ShapeDtypeStruct((IPAD, DL), _i32),
    ),
    mesh=_mesh,
    compiler_params=_sc_params,
    scratch_types=[
        pltpu.VMEM((IPT,), _i32),
        pltpu.VMEM((IPT, DL), _f32),
        pltpu.VMEM((IPT, DL), _f32),
        pltpu.VMEM((IPT, DL), _f32),
        pltpu.VMEM((IPT, DL), _f32),
        pltpu.VMEM((IPT, DL), _i32),
    ],
)


def _mm1_body(x_ref, w_ref, d0_ref, d1_ref, g1_ref, dv_ref, dv16_ref):
    deg = d0_ref[...] + d1_ref[...] + 1.0
    dv = lax.rsqrt(jnp.maximum(deg, 1.0))
    h = jnp.dot(x_ref[...], w_ref[...], preferred_element_type=_f32)
    g1_ref[...] = h * dv
    dv_ref[...] = dv
    dv16_ref[...] = jnp.broadcast_to(dv, (ROWB, DL))


def _mm1_call(x_p, W1, deg0, deg1):
    return pl.pallas_call(
        _mm1_body,
        grid=(NPAD // ROWB,),
        in_specs=[
            pl.BlockSpec((ROWB, D), lambda i: (i, 0)),
            pl.BlockSpec((D, D), lambda i: (0, 0)),
            pl.BlockSpec((ROWB, 1), lambda i: (i, 0)),
            pl.BlockSpec((ROWB, 1), lambda i: (i, 0)),
        ],
        out_specs=[
            pl.BlockSpec((ROWB, D), lambda i: (i, 0)),
            pl.BlockSpec((ROWB, 1), lambda i: (i, 0)),
            pl.BlockSpec((ROWB, DL), lambda i: (i, 0)),
        ],
        out_shape=[
            jax.ShapeDtypeStruct((NPAD, D), _f32),
            jax.ShapeDtypeStruct((NPAD, 1), _f32),
            jax.ShapeDtypeStruct((NPAD, DL), _f32),
        ],
    )(x_p, W1, deg0, deg1)


def _mm2_body(aa_ref, ab_ref, g1_ref, dv_ref, b1_ref, w2_ref, g2_ref):
    dv = dv_ref[...]
    t = dv * (aa_ref[...] + ab_ref[...] + g1_ref[...]) + b1_ref[...]
    h2 = jnp.maximum(t, 0.0)
    g2_ref[...] = jnp.dot(h2, w2_ref[...], preferred_element_type=_f32) * dv


def _mm2_call(acc1a, acc1b, g1, dinv, b1r, W2):
    return pl.pallas_call(
        _mm2_body,
        grid=(NPAD // ROWB,),
        in_specs=[
            pl.BlockSpec((ROWB, D), lambda i: (i, 0)),
            pl.BlockSpec((ROWB, D), lambda i: (i, 0)),
            pl.BlockSpec((ROWB, D), lambda i: (i, 0)),
            pl.BlockSpec((ROWB, 1), lambda i: (i, 0)),
            pl.BlockSpec((1, D), lambda i: (0, 0)),
            pl.BlockSpec((D, DL), lambda i: (0, 0)),
        ],
        out_specs=pl.BlockSpec((ROWB, DL), lambda i: (i, 0)),
        out_shape=jax.ShapeDtypeStruct((NPAD, DL), _f32),
    )(acc1a, acc1b, g1, dinv, b1r, W2)


def _loss_body(ya_ref, yb_ref, yg_ref, dv_ref, lab_ref, b2_ref, y_ref, loss_ref):
    y = dv_ref[...] * (ya_ref[...] + yb_ref[...] + yg_ref[...]) + b2_ref[...]
    m = jnp.max(y, axis=1, keepdims=True)
    lse = m + jnp.log(jnp.sum(jnp.exp(y - m), axis=1, keepdims=True))
    onehot = lax.broadcasted_iota(_i32, (IPAD, DL), 1) == lab_ref[...]
    ylab = jnp.sum(jnp.where(onehot, y, 0.0), axis=1, keepdims=True)
    nll = lse - ylab
    valid = lax.broadcasted_iota(_i32, (IPAD, 1), 0) < NIDS
    loss = jnp.sum(jnp.where(valid, nll, 0.0)) / float(NIDS)
    y_ref[...] = y
    loss_ref[...] = jnp.reshape(loss, (1, 1))


def _loss_call(ya, yb, yg, ydv, ylab, b2r):
    return pl.pallas_call(
        _loss_body,
        grid=(1,),
        in_specs=[
            pl.BlockSpec((IPAD, DL), lambda i: (0, 0)),
            pl.BlockSpec((IPAD, DL), lambda i: (0, 0)),
            pl.BlockSpec((IPAD, DL), lambda i: (0, 0)),
            pl.BlockSpec((IPAD, DL), lambda i: (0, 0)),
            pl.BlockSpec((IPAD, DL), lambda i: (0, 0)),
            pl.BlockSpec((1, DL), lambda i: (0, 0)),
        ],
        out_specs=[
            pl.BlockSpec((IPAD, DL), lambda i: (0, 0)),
            pl.BlockSpec((1, 1), lambda i: (0, 0)),
        ],
        out_shape=[
            jax.ShapeDtypeStruct((IPAD, DL), _f32),
            jax.ShapeDtypeStruct((1, 1), _f32),
        ],
    )(ya, yb, yg, ydv, ylab, b2r)


def kernel(x, edge_index, node_ids, label_inds, W1, b1, W2, b2):
    src = edge_index[0].astype(_i32)
    dst = edge_index[1].astype(_i32)
    src3 = jnp.concatenate(
        [src, jnp.zeros((EPAD - E,), _i32)]).reshape(NW, NCHUNK, CH)
    dst3 = jnp.concatenate(
        [dst, jnp.full((EPAD - E,), N, _i32)]).reshape(NW, NCHUNK, CH)
    x_p = jnp.pad(x, ((0, NPAD - N), (0, 0)))
    z_nd = jnp.zeros((NPAD, D), _f32)
    z_16 = jnp.zeros((NPAD, DL), _f32)
    ones_16 = jnp.ones((CH, DL), _f32)
    ids2 = jnp.pad(node_ids.astype(_i32), (0, IPAD - NIDS)).reshape(NW, IPT)
    lab16 = jnp.broadcast_to(
        jnp.pad(label_inds.astype(_i32), (0, NPAD - N))[:, None], (NPAD, DL))

    deg = _deg_call(dst3, z_16, ones_16)
    g1, dinv, dinv16 = _mm1_call(x_p, W1, deg[0, :, :1], deg[1, :, :1])

    acc1 = _acc128_call(src3, dst3, g1, z_nd)
    g2 = _mm2_call(acc1[0], acc1[1], g1, dinv, b1.reshape(1, D), W2)

    acc2 = _acc16_call(src3, dst3, g2, z_16)
    ya, yb, yg, ydv, ylab = _head_call(
        ids2, acc2[0], acc2[1], g2, dinv16, lab16)
    y2d, loss = _loss_call(ya, yb, yg, ydv, ylab, b2.reshape(1, DL))
    return (loss[0, 0], y2d[:NIDS])
